# baseline (speedup 1.0000x reference)
import functools

import jax
import jax.numpy as jnp
from jax import lax
from jax.experimental import pallas as pl
from jax.experimental.pallas import tpu as pltpu
from jax.experimental.pallas import tpu_sc as plsc

_NUM_HEADS = 8
_MAX_SEQ_LEN = 4096
_HEAD_DIM = 128
_TOTAL_SEQ_LEN = 32768
_BATCH = 16

_BLK_T = 2048
_SC_TOK = 4096
_SC_BLKS = _SC_TOK // _BLK_T
_TC_TOK = _TOTAL_SEQ_LEN - _SC_TOK
_NUM_BLK = _TC_TOK // _BLK_T

_NC = 2
_NS = 16
_NW = _NC * _NS
_TPT = _SC_TOK // _NW
_NGRP = _TPT // 16


def _splat(v, b):
    return jnp.take_along_axis(v, jnp.full((16,), b, jnp.int32), axis=0)


def _lane_sum(v):
    iota = lax.iota(jnp.int32, 16)
    for s in (8, 4, 2, 1):
        v = v + jnp.take_along_axis(v, jnp.bitwise_xor(iota, s), axis=0)
    return v


def _sc_body(o_hbm, do_hbm, lo_hbm, hi_hbm, out_hbm,
             lo_v, hi_v, o_buf, do_buf, res_v):
    cid = lax.axis_index("c")
    sid = lax.axis_index("s")
    wid = cid * _NS + sid
    base = wid * _TPT

    pltpu.sync_copy(lo_hbm, lo_v)
    pltpu.sync_copy(hi_hbm, hi_v)
    begin = lo_v[...]
    end = hi_v[...]
    stop = jnp.minimum(end, begin + _MAX_SEQ_LEN)
    beg_s = [_splat(begin, b) for b in range(_BATCH)]
    stop_s = [_splat(stop, b) for b in range(_BATCH)]

    iota = lax.iota(jnp.int32, 16)
    onef = jnp.full((16,), 1.0, jnp.float32)
    zerof = jnp.full((16,), 0.0, jnp.float32)

    def head(h, _unused):
        pltpu.sync_copy(o_hbm.at[h, pl.ds(base * _HEAD_DIM, _TPT * _HEAD_DIM)],
                        o_buf)
        pltpu.sync_copy(do_hbm.at[h, pl.ds(base * _HEAD_DIM, _TPT * _HEAD_DIM)],
                        do_buf)

        @plsc.parallel_loop(0, _NGRP, unroll=2)
        def _group(g):
            gb = g * (16 * _HEAD_DIM)
            res = zerof
            for k in range(16):
                row = k * _HEAD_DIM
                acc = (o_buf[pl.ds(gb + row, 16)]
                       * do_buf[pl.ds(gb + row, 16)])
                for j in range(1, _HEAD_DIM // 16):
                    off = row + j * 16
                    acc = acc + (o_buf[pl.ds(gb + off, 16)]
                                 * do_buf[pl.ds(gb + off, 16)])
                res = jnp.where(iota == k, _lane_sum(acc), res)
            tv = base + g * 16 + iota
            mk = zerof
            for b in range(_BATCH):
                inb = jnp.where(tv >= beg_s[b], onef, zerof) * jnp.where(
                    tv < stop_s[b], onef, zerof)
                mk = jnp.maximum(mk, inb)
            res_v[pl.ds(g * 16, 16)] = res * mk
        pltpu.sync_copy(res_v, out_hbm.at[h, pl.ds(base, _TPT)])
        return _unused

    lax.fori_loop(0, _NUM_HEADS, head, 0, unroll=False)


_sc_delta = functools.partial(
    pl.kernel,
    out_type=jax.ShapeDtypeStruct((_NUM_HEADS, _SC_TOK), jnp.float32),
    mesh=plsc.VectorSubcoreMesh(core_axis_name="c", subcore_axis_name="s",
                                num_cores=_NC, num_subcores=_NS),
    scratch_types=[
        pltpu.VMEM((_BATCH,), jnp.int32),
        pltpu.VMEM((_BATCH,), jnp.int32),
        pltpu.VMEM((_TPT * _HEAD_DIM,), jnp.float32),
        pltpu.VMEM((_TPT * _HEAD_DIM,), jnp.float32),
        pltpu.VMEM((_TPT,), jnp.float32),
    ],
)(_sc_body)


def _tc_body(offs_ref, o_ref, do_ref, out_ref):
    i = pl.program_id(0)
    prod = (o_ref[...] * do_ref[...]).reshape(_NUM_HEADS * _BLK_T, _HEAD_DIM)
    ones = jnp.ones((_HEAD_DIM, 128), dtype=jnp.float32)
    red = jax.lax.dot_general(
        prod, ones, (((1,), (0,)), ((), ())),
        preferred_element_type=jnp.float32)[:, :1]
    red = red.reshape(_NUM_HEADS, _BLK_T)

    t = (i + _SC_BLKS) * _BLK_T + jax.lax.broadcasted_iota(
        jnp.int32, (_NUM_HEADS, _BLK_T), 1)
    valid = jnp.zeros((_NUM_HEADS, _BLK_T), dtype=jnp.bool_)
    for b in range(_BATCH):
        begin = offs_ref[b]
        stop = jnp.minimum(offs_ref[b + 1], begin + _MAX_SEQ_LEN)
        valid = valid | ((t >= begin) & (t < stop))
    out_ref[...] = jnp.where(valid, red, 0.0)


def kernel(o, do, o_offset):
    sc_out = _sc_delta(o.reshape(_NUM_HEADS, -1), do.reshape(_NUM_HEADS, -1),
                       o_offset[:_BATCH], o_offset[1:_BATCH + 1])

    grid_spec = pltpu.PrefetchScalarGridSpec(
        num_scalar_prefetch=1,
        grid=(_NUM_BLK,),
        in_specs=[
            pl.BlockSpec((_NUM_HEADS, _BLK_T, _HEAD_DIM),
                         lambda i, offs: (0, i + _SC_BLKS, 0)),
            pl.BlockSpec((_NUM_HEADS, _BLK_T, _HEAD_DIM),
                         lambda i, offs: (0, i + _SC_BLKS, 0)),
        ],
        out_specs=pl.BlockSpec((_NUM_HEADS, _BLK_T), lambda i, offs: (0, i)),
    )
    tc_out = pl.pallas_call(
        _tc_body,
        grid_spec=grid_spec,
        out_shape=jax.ShapeDtypeStruct((_NUM_HEADS, _TC_TOK), jnp.float32),
    )(o_offset, o, do)
    return jnp.concatenate([sc_out, tc_out], axis=1)


# speedup vs baseline: 3.1046x; 3.1046x over previous
import functools

import jax
import jax.numpy as jnp
from jax import lax
from jax.experimental import pallas as pl
from jax.experimental.pallas import tpu as pltpu
from jax.experimental.pallas import tpu_sc as plsc

_NUM_HEADS = 8
_MAX_SEQ_LEN = 4096
_HEAD_DIM = 128
_TOTAL_SEQ_LEN = 32768
_BATCH = 16

_BLK_T = 2048
_SC_TOK = 2048
_SC_BLKS = _SC_TOK // _BLK_T
_TC_TOK = _TOTAL_SEQ_LEN - _SC_TOK
_NUM_BLK = _TC_TOK // _BLK_T

_NC = 2
_NS = 16
_NW = _NC * _NS
_TPT = _SC_TOK // _NW
_NGRP = _TPT // 16


def _splat(v, b):
    return jnp.take_along_axis(v, jnp.full((16,), b, jnp.int32), axis=0)


def _lane_sum(v):
    iota = lax.iota(jnp.int32, 16)
    for s in (8, 4, 2, 1):
        v = v + jnp.take_along_axis(v, jnp.bitwise_xor(iota, s), axis=0)
    return v


def _sc_body(o_hbm, do_hbm, lo_hbm, hi_hbm, out_hbm,
             lo_v, hi_v, o_buf, do_buf, res_v):
    cid = lax.axis_index("c")
    sid = lax.axis_index("s")
    wid = cid * _NS + sid
    base = wid * _TPT

    pltpu.sync_copy(lo_hbm, lo_v)
    pltpu.sync_copy(hi_hbm, hi_v)
    begin = lo_v[...]
    end = hi_v[...]
    stop = jnp.minimum(end, begin + _MAX_SEQ_LEN)
    beg_s = [_splat(begin, b) for b in range(_BATCH)]
    stop_s = [_splat(stop, b) for b in range(_BATCH)]

    iota = lax.iota(jnp.int32, 16)
    onef = jnp.full((16,), 1.0, jnp.float32)
    zerof = jnp.full((16,), 0.0, jnp.float32)

    def head(h, _unused):
        pltpu.sync_copy(o_hbm.at[h, pl.ds(base, _TPT), :], o_buf)
        pltpu.sync_copy(do_hbm.at[h, pl.ds(base, _TPT), :], do_buf)

        @plsc.parallel_loop(0, _NGRP, unroll=2)
        def _group(g):
            res = zerof
            for k in range(16):
                t_row = g * 16 + k
                acc = o_buf[t_row, pl.ds(0, 16)] * do_buf[t_row, pl.ds(0, 16)]
                for j in range(1, _HEAD_DIM // 16):
                    acc = acc + (o_buf[t_row, pl.ds(j * 16, 16)]
                                 * do_buf[t_row, pl.ds(j * 16, 16)])
                res = jnp.where(iota == k, _lane_sum(acc), res)
            tv = base + g * 16 + iota
            mk = zerof
            for b in range(_BATCH):
                inb = jnp.where(tv >= beg_s[b], onef, zerof) * jnp.where(
                    tv < stop_s[b], onef, zerof)
                mk = jnp.maximum(mk, inb)
            res_v[pl.ds(g * 16, 16)] = res * mk
        pltpu.sync_copy(res_v, out_hbm.at[h, pl.ds(base, _TPT)])
        return _unused

    lax.fori_loop(0, _NUM_HEADS, head, 0, unroll=False)


_sc_delta = functools.partial(
    pl.kernel,
    out_type=jax.ShapeDtypeStruct((_NUM_HEADS, _SC_TOK), jnp.float32),
    mesh=plsc.VectorSubcoreMesh(core_axis_name="c", subcore_axis_name="s",
                                num_cores=_NC, num_subcores=_NS),
    scratch_types=[
        pltpu.VMEM((_BATCH,), jnp.int32),
        pltpu.VMEM((_BATCH,), jnp.int32),
        pltpu.VMEM((_TPT, _HEAD_DIM), jnp.float32),
        pltpu.VMEM((_TPT, _HEAD_DIM), jnp.float32),
        pltpu.VMEM((_TPT,), jnp.float32),
    ],
)(_sc_body)


def _tc_body(offs_ref, o_ref, do_ref, out_ref):
    i = pl.program_id(0)
    prod = (o_ref[...] * do_ref[...]).reshape(_NUM_HEADS * _BLK_T, _HEAD_DIM)
    ones = jnp.ones((_HEAD_DIM, 128), dtype=jnp.float32)
    red = jax.lax.dot_general(
        prod, ones, (((1,), (0,)), ((), ())),
        preferred_element_type=jnp.float32)[:, :1]
    red = red.reshape(_NUM_HEADS, _BLK_T)

    t = (i + _SC_BLKS) * _BLK_T + jax.lax.broadcasted_iota(
        jnp.int32, (_NUM_HEADS, _BLK_T), 1)
    valid = jnp.zeros((_NUM_HEADS, _BLK_T), dtype=jnp.bool_)
    for b in range(_BATCH):
        begin = offs_ref[b]
        stop = jnp.minimum(offs_ref[b + 1], begin + _MAX_SEQ_LEN)
        valid = valid | ((t >= begin) & (t < stop))
    out_ref[...] = jnp.where(valid, red, 0.0)


def kernel(o, do, o_offset):
    sc_out = _sc_delta(o, do, o_offset[:_BATCH], o_offset[1:_BATCH + 1])

    grid_spec = pltpu.PrefetchScalarGridSpec(
        num_scalar_prefetch=1,
        grid=(_NUM_BLK,),
        in_specs=[
            pl.BlockSpec((_NUM_HEADS, _BLK_T, _HEAD_DIM),
                         lambda i, offs: (0, i + _SC_BLKS, 0)),
            pl.BlockSpec((_NUM_HEADS, _BLK_T, _HEAD_DIM),
                         lambda i, offs: (0, i + _SC_BLKS, 0)),
        ],
        out_specs=pl.BlockSpec((_NUM_HEADS, _BLK_T), lambda i, offs: (0, i)),
    )
    tc_out = pl.pallas_call(
        _tc_body,
        grid_spec=grid_spec,
        out_shape=jax.ShapeDtypeStruct((_NUM_HEADS, _TC_TOK), jnp.float32),
    )(o_offset, o, do)
    return jnp.concatenate([sc_out, tc_out], axis=1)


# speedup vs baseline: 3.7649x; 1.2127x over previous
import jax
import jax.numpy as jnp
from jax.experimental import pallas as pl
from jax.experimental.pallas import tpu as pltpu

_NUM_HEADS = 8
_MAX_SEQ_LEN = 4096
_HEAD_DIM = 128
_TOTAL_SEQ_LEN = 32768
_BATCH = 16

_BLK_T = 2048
_NUM_BLK = _TOTAL_SEQ_LEN // _BLK_T


def _tc_body(offs_ref, o_ref, do_ref, out_ref):
    i = pl.program_id(0)
    prod = (o_ref[...] * do_ref[...]).reshape(_NUM_HEADS * _BLK_T, _HEAD_DIM)
    ones = jnp.ones((_HEAD_DIM, 128), dtype=jnp.float32)
    red = jax.lax.dot_general(
        prod, ones, (((1,), (0,)), ((), ())),
        preferred_element_type=jnp.float32)[:, :1]
    red = red.reshape(_NUM_HEADS, _BLK_T)

    t = i * _BLK_T + jax.lax.broadcasted_iota(jnp.int32, (_NUM_HEADS, _BLK_T), 1)
    valid = jnp.zeros((_NUM_HEADS, _BLK_T), dtype=jnp.bool_)
    for b in range(_BATCH):
        begin = offs_ref[b]
        stop = jnp.minimum(offs_ref[b + 1], begin + _MAX_SEQ_LEN)
        valid = valid | ((t >= begin) & (t < stop))
    out_ref[...] = jnp.where(valid, red, 0.0)


def kernel(o, do, o_offset):
    grid_spec = pltpu.PrefetchScalarGridSpec(
        num_scalar_prefetch=1,
        grid=(_NUM_BLK,),
        in_specs=[
            pl.BlockSpec((_NUM_HEADS, _BLK_T, _HEAD_DIM), lambda i, offs: (0, i, 0)),
            pl.BlockSpec((_NUM_HEADS, _BLK_T, _HEAD_DIM), lambda i, offs: (0, i, 0)),
        ],
        out_specs=pl.BlockSpec((_NUM_HEADS, _BLK_T), lambda i, offs: (0, i)),
    )
    return pl.pallas_call(
        _tc_body,
        grid_spec=grid_spec,
        out_shape=jax.ShapeDtypeStruct((_NUM_HEADS, _TOTAL_SEQ_LEN), jnp.float32),
    )(o_offset, o, do)
